# baseline (speedup 1.0000x reference)
import functools

import jax
import jax.numpy as jnp
from jax import lax
from jax.experimental import pallas as pl
from jax.experimental.pallas import tpu as pltpu
from jax.experimental.pallas import tpu_sc as plsc

N = 10000
NPAD = 10240
D = 128
E = 320000
NG = 64
NW = 32
EW = E // NW
CH = 64
C = 160
EWPAD = C * CH
TRASH = NPAD
ACC_ROWS = NPAD + 16
ZROWS = ACC_ROWS // 16
WROWS = NPAD // 16

_mesh = plsc.VectorSubcoreMesh(core_axis_name="c", subcore_axis_name="s")


def _deg_body(dst_hbm, ones_hbm, zeros_hbm, deg_out, acc, dsbuf, vones,
              sem0, sem1):
    cc = lax.axis_index("c")
    s = lax.axis_index("s")
    w = s * 2 + cc
    pltpu.sync_copy(zeros_hbm, acc.at[pl.ds(s * ZROWS, ZROWS)])
    pltpu.sync_copy(ones_hbm, vones)
    pltpu.sync_copy(dst_hbm.at[w], dsbuf)
    plsc.subcore_barrier()

    def body2(i, carry):
        c = i * 2
        a1 = pltpu.async_copy(vones, acc.at[dsbuf.at[c]], sem0, add=True)
        a2 = pltpu.async_copy(vones, acc.at[dsbuf.at[c + 1]], sem1, add=True)
        a1.wait()
        a2.wait()
        return carry

    lax.fori_loop(0, C // 2, body2, 0)
    plsc.subcore_barrier()
    pltpu.sync_copy(acc.at[pl.ds(s * WROWS, WROWS)],
                    deg_out.at[cc, pl.ds(s * WROWS, WROWS)])


def _sc_deg(dst3, ones128, zeros128):
    return pl.kernel(
        _deg_body,
        out_type=jax.ShapeDtypeStruct((2, NPAD, D), jnp.float32),
        mesh=_mesh,
        scratch_types=[
            pltpu.VMEM_SHARED((ACC_ROWS, D), jnp.float32),
            pltpu.VMEM((C, CH), jnp.int32),
            pltpu.VMEM((CH, D), jnp.float32),
            pltpu.SemaphoreType.DMA,
            pltpu.SemaphoreType.DMA,
        ],
    )(dst3, ones128, zeros128)


GC = 40
NGRP = C // GC


def _scatter_body(hp_hbm, sd_hbm, zeros_hbm, out, acc, idxb, rows0, rows1,
                  gsem0, gsem1, isem0, isem1):
    cc = lax.axis_index("c")
    s = lax.axis_index("s")
    w = s * 2 + cc
    isems = (isem0, isem1)
    pltpu.sync_copy(zeros_hbm, acc.at[pl.ds(s * ZROWS, ZROWS)])
    pltpu.sync_copy(sd_hbm.at[w, 0], idxb.at[0])
    plsc.subcore_barrier()

    pltpu.async_copy(hp_hbm.at[idxb.at[0, 0, 0]], rows0, gsem0).wait()

    for g in range(NGRP):
        par = g % 2
        if g + 1 < NGRP:
            ai = pltpu.async_copy(sd_hbm.at[w, g + 1], idxb.at[1 - par],
                                  isems[1 - par])

        def pair(j, carry, par=par):
            c0 = j * 2
            g1 = pltpu.async_copy(hp_hbm.at[idxb.at[par, c0 + 1, 0]],
                                  rows1, gsem1)
            pltpu.sync_copy(rows0, acc.at[idxb.at[par, c0, 1]], add=True)
            g1.wait()
            g2 = pltpu.async_copy(hp_hbm.at[idxb.at[par, c0 + 2, 0]],
                                  rows0, gsem0)
            pltpu.sync_copy(rows1, acc.at[idxb.at[par, c0 + 1, 1]], add=True)
            g2.wait()
            return carry

        lax.fori_loop(0, GC // 2 - 1, pair, 0)
        g1 = pltpu.async_copy(hp_hbm.at[idxb.at[par, GC - 1, 0]],
                              rows1, gsem1)
        pltpu.sync_copy(rows0, acc.at[idxb.at[par, GC - 2, 1]], add=True)
        g1.wait()
        if g + 1 < NGRP:
            ai.wait()
            g2 = pltpu.async_copy(hp_hbm.at[idxb.at[1 - par, 0, 0]],
                                  rows0, gsem0)
        pltpu.sync_copy(rows1, acc.at[idxb.at[par, GC - 1, 1]], add=True)
        if g + 1 < NGRP:
            g2.wait()

    plsc.subcore_barrier()
    pltpu.sync_copy(acc.at[pl.ds(s * WROWS, WROWS)],
                    out.at[cc, pl.ds(s * WROWS, WROWS)])


def _sc_scatter(hp, sd, zeros128):
    return pl.kernel(
        _scatter_body,
        out_type=jax.ShapeDtypeStruct((2, NPAD, D), jnp.float32),
        mesh=_mesh,
        scratch_types=[
            pltpu.VMEM_SHARED((ACC_ROWS, D), jnp.float32),
            pltpu.VMEM((2, GC, 2, CH), jnp.int32),
            pltpu.VMEM((CH, D), jnp.float32),
            pltpu.VMEM((CH, D), jnp.float32),
            pltpu.SemaphoreType.DMA,
            pltpu.SemaphoreType.DMA,
            pltpu.SemaphoreType.DMA,
            pltpu.SemaphoreType.DMA,
        ],
    )(hp, sd, zeros128)


BLK = 1024
GRID = NPAD // BLK


def _dinv_of(degp):
    deg = degp[0, :, 0] + degp[1, :, 0] + 1.0
    return lax.rsqrt(deg)


def _h0_body(x_ref, w_ref, degp_ref, out_ref):
    dinv = _dinv_of(degp_ref[...])
    h = jnp.dot(x_ref[...], w_ref[...], preferred_element_type=jnp.float32)
    out_ref[...] = h * dinv[:, None]


def _tc_h0(xpad, W0, degp):
    return pl.pallas_call(
        _h0_body,
        grid=(GRID,),
        in_specs=[
            pl.BlockSpec((BLK, D), lambda i: (i, 0)),
            pl.BlockSpec((D, D), lambda i: (0, 0)),
            pl.BlockSpec((2, BLK, D), lambda i: (0, i, 0)),
        ],
        out_specs=pl.BlockSpec((BLK, D), lambda i: (i, 0)),
        out_shape=jax.ShapeDtypeStruct((NPAD, D), jnp.float32),
    )(xpad, W0, degp)


def _mid_body(acc_ref, hp_ref, degp_ref, b_ref, w_ref, out_ref):
    dinv = _dinv_of(degp_ref[...])
    t = dinv[:, None] * (acc_ref[0] + acc_ref[1] + hp_ref[...]) + b_ref[...]
    t = jnp.maximum(t, 0.0)
    h = jnp.dot(t, w_ref[...], preferred_element_type=jnp.float32)
    out_ref[...] = h * dinv[:, None]


def _tc_mid(acc, hp, degp, b, W):
    return pl.pallas_call(
        _mid_body,
        grid=(GRID,),
        in_specs=[
            pl.BlockSpec((2, BLK, D), lambda i: (0, i, 0)),
            pl.BlockSpec((BLK, D), lambda i: (i, 0)),
            pl.BlockSpec((2, BLK, D), lambda i: (0, i, 0)),
            pl.BlockSpec((1, D), lambda i: (0, 0)),
            pl.BlockSpec((D, D), lambda i: (0, 0)),
        ],
        out_specs=pl.BlockSpec((BLK, D), lambda i: (i, 0)),
        out_shape=jax.ShapeDtypeStruct((NPAD, D), jnp.float32),
    )(acc, hp, degp, b, W)


def _last_body(acc_ref, hp_ref, degp_ref, b_ref, out_ref):
    dinv = _dinv_of(degp_ref[...])
    out_ref[...] = (dinv[:, None] * (acc_ref[0] + acc_ref[1] + hp_ref[...])
                    + b_ref[...])


def _tc_last(acc, hp, degp, b):
    return pl.pallas_call(
        _last_body,
        grid=(GRID,),
        in_specs=[
            pl.BlockSpec((2, BLK, D), lambda i: (0, i, 0)),
            pl.BlockSpec((BLK, D), lambda i: (i, 0)),
            pl.BlockSpec((2, BLK, D), lambda i: (0, i, 0)),
            pl.BlockSpec((1, D), lambda i: (0, 0)),
        ],
        out_specs=pl.BlockSpec((BLK, D), lambda i: (i, 0)),
        out_shape=jax.ShapeDtypeStruct((NPAD, D), jnp.float32),
    )(acc, hp, degp, b)


def _poolhead_body(hf_ref, batch_ref, fc1w_ref, fc1b_ref, fc2w_ref,
                   fc2b_ref, outw_ref, outb_ref, y_ref):
    seg = lax.broadcasted_iota(jnp.int32, (NG, NPAD), 0)
    m = (seg == batch_ref[...]).astype(jnp.float32)
    sums = jnp.dot(m, hf_ref[...], preferred_element_type=jnp.float32)
    cnt = jnp.sum(m, axis=1, keepdims=True)
    g = sums / jnp.maximum(cnt, 1.0)
    y1 = jnp.maximum(
        jnp.dot(g, fc1w_ref[...], preferred_element_type=jnp.float32)
        + fc1b_ref[...], 0.0)
    y2 = jnp.sum(y1 * fc2w_ref[...], axis=1, keepdims=True) + fc2b_ref[0, 0]
    y = y2 * outw_ref[0, 0] + outb_ref[0, 0]
    y_ref[...] = jnp.broadcast_to(y, (NG, D))


def _tc_poolhead(hf, batchp, fc1_w, fc1_b, fc2_w, fc2_b, out_w, out_b):
    return pl.pallas_call(
        _poolhead_body,
        out_shape=jax.ShapeDtypeStruct((NG, D), jnp.float32),
    )(hf, batchp, fc1_w, fc1_b, fc2_w, fc2_b, out_w, out_b)


@jax.jit
def _run(x, edge_index, batch, W0, b0, W1, b1, W2, b2,
         fc1_w, fc1_b, fc2_w, fc2_b, out_w, out_b):
    src2 = jnp.pad(edge_index[0].reshape(NW, EW), ((0, 0), (0, EWPAD - EW)))
    dst2 = jnp.pad(edge_index[1].reshape(NW, EW), ((0, 0), (0, EWPAD - EW)),
                   constant_values=TRASH)
    s3 = src2.reshape(NW, C, CH)
    d3 = dst2.reshape(NW, C, CH)
    sd = jnp.stack([s3, d3], axis=2).reshape(NW, NGRP, GC, 2, CH)

    xpad = jnp.pad(x, ((0, NPAD - N), (0, 0)))
    batchp = jnp.pad(batch, (0, NPAD - N),
                     constant_values=NG).reshape(1, NPAD)

    ones128 = jnp.ones((CH, D), jnp.float32)
    zeros128 = jnp.zeros((ZROWS, D), jnp.float32)

    degp = _sc_deg(d3, ones128, zeros128)

    h0p = _tc_h0(xpad, W0, degp)
    acc0 = _sc_scatter(h0p, sd, zeros128)
    h1p = _tc_mid(acc0, h0p, degp, b0.reshape(1, D), W1)
    acc1 = _sc_scatter(h1p, sd, zeros128)
    h2p = _tc_mid(acc1, h1p, degp, b1.reshape(1, D), W2)
    acc2 = _sc_scatter(h2p, sd, zeros128)
    hf = _tc_last(acc2, h2p, degp, b2.reshape(1, D))

    y = _tc_poolhead(hf, batchp, fc1_w, fc1_b.reshape(1, NG),
                     fc2_w.reshape(1, NG), fc2_b.reshape(1, 1),
                     out_w.reshape(1, 1), out_b.reshape(1, 1))
    return y[:, :1]


def kernel(x, edge_index, batch, W0, b0, W1, b1, W2, b2,
           fc1_w, fc1_b, fc2_w, fc2_b, out_w, out_b):
    return _run(x, edge_index, batch, W0, b0, W1, b1, W2, b2,
                fc1_w, fc1_b, fc2_w, fc2_b, out_w, out_b)


# speedup vs baseline: 1.0880x; 1.0880x over previous
import functools

import jax
import jax.numpy as jnp
from jax import lax
from jax.experimental import pallas as pl
from jax.experimental.pallas import tpu as pltpu
from jax.experimental.pallas import tpu_sc as plsc

N = 10000
NPAD = 10240
D = 128
E = 320000
NG = 64
NW = 32
EW = E // NW
CH = 128
C = 80
EWPAD = C * CH
TRASH = NPAD
ACC_ROWS = NPAD + 16
ZROWS = ACC_ROWS // 16
WROWS = NPAD // 16

_mesh = plsc.VectorSubcoreMesh(core_axis_name="c", subcore_axis_name="s")


def _deg_body(dst_hbm, ones_hbm, zeros_hbm, deg_out, acc, dsbuf, vones,
              sem0, sem1):
    cc = lax.axis_index("c")
    s = lax.axis_index("s")
    w = s * 2 + cc
    pltpu.sync_copy(zeros_hbm, acc.at[pl.ds(s * ZROWS, ZROWS)])
    pltpu.sync_copy(ones_hbm, vones)
    pltpu.sync_copy(dst_hbm.at[w], dsbuf)
    plsc.subcore_barrier()

    def body2(i, carry):
        c = i * 2
        a1 = pltpu.async_copy(vones, acc.at[dsbuf.at[c]], sem0, add=True)
        a2 = pltpu.async_copy(vones, acc.at[dsbuf.at[c + 1]], sem1, add=True)
        a1.wait()
        a2.wait()
        return carry

    lax.fori_loop(0, C // 2, body2, 0)
    plsc.subcore_barrier()
    pltpu.sync_copy(acc.at[pl.ds(s * WROWS, WROWS)],
                    deg_out.at[cc, pl.ds(s * WROWS, WROWS)])


def _sc_deg(dst3, ones128, zeros128):
    return pl.kernel(
        _deg_body,
        out_type=jax.ShapeDtypeStruct((2, NPAD, D), jnp.float32),
        mesh=_mesh,
        scratch_types=[
            pltpu.VMEM_SHARED((ACC_ROWS, D), jnp.float32),
            pltpu.VMEM((C, CH), jnp.int32),
            pltpu.VMEM((CH, D), jnp.float32),
            pltpu.SemaphoreType.DMA,
            pltpu.SemaphoreType.DMA,
        ],
    )(dst3, ones128, zeros128)


GC = 10
NGRP = C // GC


def _scatter_body(hp_hbm, sd_hbm, zeros_hbm, out, acc, idxb, rows0, rows1,
                  gsem0, gsem1, isem0, isem1):
    cc = lax.axis_index("c")
    s = lax.axis_index("s")
    w = s * 2 + cc
    isems = (isem0, isem1)
    pltpu.sync_copy(zeros_hbm, acc.at[pl.ds(s * ZROWS, ZROWS)])
    pltpu.sync_copy(sd_hbm.at[w, 0], idxb.at[0])
    plsc.subcore_barrier()

    pltpu.async_copy(hp_hbm.at[idxb.at[0, 0, 0]], rows0, gsem0).wait()

    for g in range(NGRP):
        par = g % 2
        if g + 1 < NGRP:
            ai = pltpu.async_copy(sd_hbm.at[w, g + 1], idxb.at[1 - par],
                                  isems[1 - par])

        def pair(j, carry, par=par):
            c0 = j * 2
            g1 = pltpu.async_copy(hp_hbm.at[idxb.at[par, c0 + 1, 0]],
                                  rows1, gsem1)
            pltpu.sync_copy(rows0, acc.at[idxb.at[par, c0, 1]], add=True)
            g1.wait()
            g2 = pltpu.async_copy(hp_hbm.at[idxb.at[par, c0 + 2, 0]],
                                  rows0, gsem0)
            pltpu.sync_copy(rows1, acc.at[idxb.at[par, c0 + 1, 1]], add=True)
            g2.wait()
            return carry

        lax.fori_loop(0, GC // 2 - 1, pair, 0)
        g1 = pltpu.async_copy(hp_hbm.at[idxb.at[par, GC - 1, 0]],
                              rows1, gsem1)
        pltpu.sync_copy(rows0, acc.at[idxb.at[par, GC - 2, 1]], add=True)
        g1.wait()
        if g + 1 < NGRP:
            ai.wait()
            g2 = pltpu.async_copy(hp_hbm.at[idxb.at[1 - par, 0, 0]],
                                  rows0, gsem0)
        pltpu.sync_copy(rows1, acc.at[idxb.at[par, GC - 1, 1]], add=True)
        if g + 1 < NGRP:
            g2.wait()

    plsc.subcore_barrier()
    pltpu.sync_copy(acc.at[pl.ds(s * WROWS, WROWS)],
                    out.at[cc, pl.ds(s * WROWS, WROWS)])


def _sc_scatter(hp, sd, zeros128):
    return pl.kernel(
        _scatter_body,
        out_type=jax.ShapeDtypeStruct((2, NPAD, D), jnp.float32),
        mesh=_mesh,
        scratch_types=[
            pltpu.VMEM_SHARED((ACC_ROWS, D), jnp.float32),
            pltpu.VMEM((2, GC, 2, CH), jnp.int32),
            pltpu.VMEM((CH, D), jnp.float32),
            pltpu.VMEM((CH, D), jnp.float32),
            pltpu.SemaphoreType.DMA,
            pltpu.SemaphoreType.DMA,
            pltpu.SemaphoreType.DMA,
            pltpu.SemaphoreType.DMA,
        ],
    )(hp, sd, zeros128)


BLK = 1024
GRID = NPAD // BLK


def _dinv_of(degp):
    deg = degp[0, :, 0] + degp[1, :, 0] + 1.0
    return lax.rsqrt(deg)


def _h0_body(x_ref, w_ref, degp_ref, out_ref):
    dinv = _dinv_of(degp_ref[...])
    h = jnp.dot(x_ref[...], w_ref[...], preferred_element_type=jnp.float32)
    out_ref[...] = h * dinv[:, None]


def _tc_h0(xpad, W0, degp):
    return pl.pallas_call(
        _h0_body,
        grid=(GRID,),
        in_specs=[
            pl.BlockSpec((BLK, D), lambda i: (i, 0)),
            pl.BlockSpec((D, D), lambda i: (0, 0)),
            pl.BlockSpec((2, BLK, D), lambda i: (0, i, 0)),
        ],
        out_specs=pl.BlockSpec((BLK, D), lambda i: (i, 0)),
        out_shape=jax.ShapeDtypeStruct((NPAD, D), jnp.float32),
    )(xpad, W0, degp)


def _mid_body(acc_ref, hp_ref, degp_ref, b_ref, w_ref, out_ref):
    dinv = _dinv_of(degp_ref[...])
    t = dinv[:, None] * (acc_ref[0] + acc_ref[1] + hp_ref[...]) + b_ref[...]
    t = jnp.maximum(t, 0.0)
    h = jnp.dot(t, w_ref[...], preferred_element_type=jnp.float32)
    out_ref[...] = h * dinv[:, None]


def _tc_mid(acc, hp, degp, b, W):
    return pl.pallas_call(
        _mid_body,
        grid=(GRID,),
        in_specs=[
            pl.BlockSpec((2, BLK, D), lambda i: (0, i, 0)),
            pl.BlockSpec((BLK, D), lambda i: (i, 0)),
            pl.BlockSpec((2, BLK, D), lambda i: (0, i, 0)),
            pl.BlockSpec((1, D), lambda i: (0, 0)),
            pl.BlockSpec((D, D), lambda i: (0, 0)),
        ],
        out_specs=pl.BlockSpec((BLK, D), lambda i: (i, 0)),
        out_shape=jax.ShapeDtypeStruct((NPAD, D), jnp.float32),
    )(acc, hp, degp, b, W)


def _last_body(acc_ref, hp_ref, degp_ref, b_ref, out_ref):
    dinv = _dinv_of(degp_ref[...])
    out_ref[...] = (dinv[:, None] * (acc_ref[0] + acc_ref[1] + hp_ref[...])
                    + b_ref[...])


def _tc_last(acc, hp, degp, b):
    return pl.pallas_call(
        _last_body,
        grid=(GRID,),
        in_specs=[
            pl.BlockSpec((2, BLK, D), lambda i: (0, i, 0)),
            pl.BlockSpec((BLK, D), lambda i: (i, 0)),
            pl.BlockSpec((2, BLK, D), lambda i: (0, i, 0)),
            pl.BlockSpec((1, D), lambda i: (0, 0)),
        ],
        out_specs=pl.BlockSpec((BLK, D), lambda i: (i, 0)),
        out_shape=jax.ShapeDtypeStruct((NPAD, D), jnp.float32),
    )(acc, hp, degp, b)


def _poolhead_body(hf_ref, batch_ref, fc1w_ref, fc1b_ref, fc2w_ref,
                   fc2b_ref, outw_ref, outb_ref, y_ref):
    seg = lax.broadcasted_iota(jnp.int32, (NG, NPAD), 0)
    m = (seg == batch_ref[...]).astype(jnp.float32)
    sums = jnp.dot(m, hf_ref[...], preferred_element_type=jnp.float32)
    cnt = jnp.sum(m, axis=1, keepdims=True)
    g = sums / jnp.maximum(cnt, 1.0)
    y1 = jnp.maximum(
        jnp.dot(g, fc1w_ref[...], preferred_element_type=jnp.float32)
        + fc1b_ref[...], 0.0)
    y2 = jnp.sum(y1 * fc2w_ref[...], axis=1, keepdims=True) + fc2b_ref[0, 0]
    y = y2 * outw_ref[0, 0] + outb_ref[0, 0]
    y_ref[...] = jnp.broadcast_to(y, (NG, D))


def _tc_poolhead(hf, batchp, fc1_w, fc1_b, fc2_w, fc2_b, out_w, out_b):
    return pl.pallas_call(
        _poolhead_body,
        out_shape=jax.ShapeDtypeStruct((NG, D), jnp.float32),
    )(hf, batchp, fc1_w, fc1_b, fc2_w, fc2_b, out_w, out_b)


@jax.jit
def _run(x, edge_index, batch, W0, b0, W1, b1, W2, b2,
         fc1_w, fc1_b, fc2_w, fc2_b, out_w, out_b):
    src2 = jnp.pad(edge_index[0].reshape(NW, EW), ((0, 0), (0, EWPAD - EW)))
    dst2 = jnp.pad(edge_index[1].reshape(NW, EW), ((0, 0), (0, EWPAD - EW)),
                   constant_values=TRASH)
    s3 = src2.reshape(NW, C, CH)
    d3 = dst2.reshape(NW, C, CH)
    sd = jnp.stack([s3, d3], axis=2).reshape(NW, NGRP, GC, 2, CH)

    xpad = jnp.pad(x, ((0, NPAD - N), (0, 0)))
    batchp = jnp.pad(batch, (0, NPAD - N),
                     constant_values=NG).reshape(1, NPAD)

    ones128 = jnp.ones((CH, D), jnp.float32)
    zeros128 = jnp.zeros((ZROWS, D), jnp.float32)

    degp = _sc_deg(d3, ones128, zeros128)

    h0p = _tc_h0(xpad, W0, degp)
    acc0 = _sc_scatter(h0p, sd, zeros128)
    h1p = _tc_mid(acc0, h0p, degp, b0.reshape(1, D), W1)
    acc1 = _sc_scatter(h1p, sd, zeros128)
    h2p = _tc_mid(acc1, h1p, degp, b1.reshape(1, D), W2)
    acc2 = _sc_scatter(h2p, sd, zeros128)
    hf = _tc_last(acc2, h2p, degp, b2.reshape(1, D))

    y = _tc_poolhead(hf, batchp, fc1_w, fc1_b.reshape(1, NG),
                     fc2_w.reshape(1, NG), fc2_b.reshape(1, 1),
                     out_w.reshape(1, 1), out_b.reshape(1, 1))
    return y[:, :1]


def kernel(x, edge_index, batch, W0, b0, W1, b1, W2, b2,
           fc1_w, fc1_b, fc2_w, fc2_b, out_w, out_b):
    return _run(x, edge_index, batch, W0, b0, W1, b1, W2, b2,
                fc1_w, fc1_b, fc2_w, fc2_b, out_w, out_b)


# speedup vs baseline: 2.7343x; 2.5131x over previous
import functools

import jax
import jax.numpy as jnp
from jax import lax
from jax.experimental import pallas as pl
from jax.experimental.pallas import tpu as pltpu
from jax.experimental.pallas import tpu_sc as plsc

N = 10000
NPAD = 10240
D = 128
E = 320000
NG = 64
NW = 32
EW = E // NW
CH = 128
C = 80
EWPAD = C * CH
TRASH = NPAD
NTRASH = 512
ACC_ROWS = NPAD + NTRASH
ZROWS = ACC_ROWS // 16
WROWS = NPAD // 16

_mesh = plsc.VectorSubcoreMesh(core_axis_name="c", subcore_axis_name="s")


def _deg_body(dst_hbm, ones_hbm, zeros_hbm, deg_out, acc, dsbuf, vones,
              sem0, sem1):
    cc = lax.axis_index("c")
    s = lax.axis_index("s")
    w = s * 2 + cc
    pltpu.sync_copy(zeros_hbm, acc.at[pl.ds(s * ZROWS, ZROWS)])
    pltpu.sync_copy(ones_hbm, vones)
    pltpu.sync_copy(dst_hbm.at[w], dsbuf)
    plsc.subcore_barrier()

    def body2(i, carry):
        c = i * 2
        a1 = pltpu.async_copy(vones, acc.at[dsbuf.at[c]], sem0, add=True)
        a2 = pltpu.async_copy(vones, acc.at[dsbuf.at[c + 1]], sem1, add=True)
        a1.wait()
        a2.wait()
        return carry

    lax.fori_loop(0, C // 2, body2, 0)
    plsc.subcore_barrier()
    pltpu.sync_copy(acc.at[pl.ds(s * WROWS, WROWS)],
                    deg_out.at[cc, pl.ds(s * WROWS, WROWS)])


def _sc_deg(dst3, ones128, zeros128):
    return pl.kernel(
        _deg_body,
        out_type=jax.ShapeDtypeStruct((2, NPAD, D), jnp.float32),
        mesh=_mesh,
        scratch_types=[
            pltpu.VMEM_SHARED((ACC_ROWS, D), jnp.float32),
            pltpu.VMEM((C, CH), jnp.int32),
            pltpu.VMEM((CH, D), jnp.float32),
            pltpu.SemaphoreType.DMA,
            pltpu.SemaphoreType.DMA,
        ],
    )(dst3, ones128, zeros128)


GC = 10
NGRP = C // GC


def _scatter_body(hp_hbm, sd_hbm, zeros_hbm, out, acc, idxb, rows0, rows1,
                  gsem0, gsem1, isem0, isem1):
    cc = lax.axis_index("c")
    s = lax.axis_index("s")
    w = s * 2 + cc
    isems = (isem0, isem1)
    pltpu.sync_copy(zeros_hbm, acc.at[pl.ds(s * ZROWS, ZROWS)])
    pltpu.sync_copy(sd_hbm.at[w, 0], idxb.at[0])
    plsc.subcore_barrier()

    pltpu.async_copy(hp_hbm.at[idxb.at[0, 0, 0]], rows0, gsem0).wait()

    for g in range(NGRP):
        par = g % 2
        if g + 1 < NGRP:
            ai = pltpu.async_copy(sd_hbm.at[w, g + 1], idxb.at[1 - par],
                                  isems[1 - par])

        def pair(j, carry, par=par):
            c0 = j * 2
            g1 = pltpu.async_copy(hp_hbm.at[idxb.at[par, c0 + 1, 0]],
                                  rows1, gsem1)
            pltpu.sync_copy(rows0, acc.at[idxb.at[par, c0, 1]], add=True)
            g1.wait()
            g2 = pltpu.async_copy(hp_hbm.at[idxb.at[par, c0 + 2, 0]],
                                  rows0, gsem0)
            pltpu.sync_copy(rows1, acc.at[idxb.at[par, c0 + 1, 1]], add=True)
            g2.wait()
            return carry

        lax.fori_loop(0, GC // 2 - 1, pair, 0)
        g1 = pltpu.async_copy(hp_hbm.at[idxb.at[par, GC - 1, 0]],
                              rows1, gsem1)
        pltpu.sync_copy(rows0, acc.at[idxb.at[par, GC - 2, 1]], add=True)
        g1.wait()
        if g + 1 < NGRP:
            ai.wait()
            g2 = pltpu.async_copy(hp_hbm.at[idxb.at[1 - par, 0, 0]],
                                  rows0, gsem0)
        pltpu.sync_copy(rows1, acc.at[idxb.at[par, GC - 1, 1]], add=True)
        if g + 1 < NGRP:
            g2.wait()

    plsc.subcore_barrier()
    pltpu.sync_copy(acc.at[pl.ds(s * WROWS, WROWS)],
                    out.at[cc, pl.ds(s * WROWS, WROWS)])


def _sc_scatter(hp, sd, zeros128):
    return pl.kernel(
        _scatter_body,
        out_type=jax.ShapeDtypeStruct((2, NPAD, D), jnp.float32),
        mesh=_mesh,
        scratch_types=[
            pltpu.VMEM_SHARED((ACC_ROWS, D), jnp.float32),
            pltpu.VMEM((2, GC, 2, CH), jnp.int32),
            pltpu.VMEM((CH, D), jnp.float32),
            pltpu.VMEM((CH, D), jnp.float32),
            pltpu.SemaphoreType.DMA,
            pltpu.SemaphoreType.DMA,
            pltpu.SemaphoreType.DMA,
            pltpu.SemaphoreType.DMA,
        ],
    )(hp, sd, zeros128)


BLK = 1024
GRID = NPAD // BLK


def _dinv_of(degp):
    deg = degp[0, :, 0] + degp[1, :, 0] + 1.0
    return lax.rsqrt(deg)


def _h0_body(x_ref, w_ref, degp_ref, out_ref):
    dinv = _dinv_of(degp_ref[...])
    h = jnp.dot(x_ref[...], w_ref[...], preferred_element_type=jnp.float32)
    out_ref[...] = h * dinv[:, None]


def _tc_h0(xpad, W0, degp):
    return pl.pallas_call(
        _h0_body,
        grid=(GRID,),
        in_specs=[
            pl.BlockSpec((BLK, D), lambda i: (i, 0)),
            pl.BlockSpec((D, D), lambda i: (0, 0)),
            pl.BlockSpec((2, BLK, D), lambda i: (0, i, 0)),
        ],
        out_specs=pl.BlockSpec((BLK, D), lambda i: (i, 0)),
        out_shape=jax.ShapeDtypeStruct((NPAD, D), jnp.float32),
    )(xpad, W0, degp)


def _mid_body(acc_ref, hp_ref, degp_ref, b_ref, w_ref, out_ref):
    dinv = _dinv_of(degp_ref[...])
    t = dinv[:, None] * (acc_ref[0] + acc_ref[1] + hp_ref[...]) + b_ref[...]
    t = jnp.maximum(t, 0.0)
    h = jnp.dot(t, w_ref[...], preferred_element_type=jnp.float32)
    out_ref[...] = h * dinv[:, None]


def _tc_mid(acc, hp, degp, b, W):
    return pl.pallas_call(
        _mid_body,
        grid=(GRID,),
        in_specs=[
            pl.BlockSpec((2, BLK, D), lambda i: (0, i, 0)),
            pl.BlockSpec((BLK, D), lambda i: (i, 0)),
            pl.BlockSpec((2, BLK, D), lambda i: (0, i, 0)),
            pl.BlockSpec((1, D), lambda i: (0, 0)),
            pl.BlockSpec((D, D), lambda i: (0, 0)),
        ],
        out_specs=pl.BlockSpec((BLK, D), lambda i: (i, 0)),
        out_shape=jax.ShapeDtypeStruct((NPAD, D), jnp.float32),
    )(acc, hp, degp, b, W)


def _last_body(acc_ref, hp_ref, degp_ref, b_ref, out_ref):
    dinv = _dinv_of(degp_ref[...])
    out_ref[...] = (dinv[:, None] * (acc_ref[0] + acc_ref[1] + hp_ref[...])
                    + b_ref[...])


def _tc_last(acc, hp, degp, b):
    return pl.pallas_call(
        _last_body,
        grid=(GRID,),
        in_specs=[
            pl.BlockSpec((2, BLK, D), lambda i: (0, i, 0)),
            pl.BlockSpec((BLK, D), lambda i: (i, 0)),
            pl.BlockSpec((2, BLK, D), lambda i: (0, i, 0)),
            pl.BlockSpec((1, D), lambda i: (0, 0)),
        ],
        out_specs=pl.BlockSpec((BLK, D), lambda i: (i, 0)),
        out_shape=jax.ShapeDtypeStruct((NPAD, D), jnp.float32),
    )(acc, hp, degp, b)


def _poolhead_body(hf_ref, batch_ref, fc1w_ref, fc1b_ref, fc2w_ref,
                   fc2b_ref, outw_ref, outb_ref, y_ref):
    seg = lax.broadcasted_iota(jnp.int32, (NG, NPAD), 0)
    m = (seg == batch_ref[...]).astype(jnp.float32)
    sums = jnp.dot(m, hf_ref[...], preferred_element_type=jnp.float32)
    cnt = jnp.sum(m, axis=1, keepdims=True)
    g = sums / jnp.maximum(cnt, 1.0)
    y1 = jnp.maximum(
        jnp.dot(g, fc1w_ref[...], preferred_element_type=jnp.float32)
        + fc1b_ref[...], 0.0)
    y2 = jnp.sum(y1 * fc2w_ref[...], axis=1, keepdims=True) + fc2b_ref[0, 0]
    y = y2 * outw_ref[0, 0] + outb_ref[0, 0]
    y_ref[...] = jnp.broadcast_to(y, (NG, D))


def _tc_poolhead(hf, batchp, fc1_w, fc1_b, fc2_w, fc2_b, out_w, out_b):
    return pl.pallas_call(
        _poolhead_body,
        out_shape=jax.ShapeDtypeStruct((NG, D), jnp.float32),
    )(hf, batchp, fc1_w, fc1_b, fc2_w, fc2_b, out_w, out_b)


@jax.jit
def _run(x, edge_index, batch, W0, b0, W1, b1, W2, b2,
         fc1_w, fc1_b, fc2_w, fc2_b, out_w, out_b):
    npad = EWPAD - EW
    wcol = jnp.arange(NW, dtype=jnp.int32)[:, None]
    jrow = jnp.arange(npad, dtype=jnp.int32)[None, :]
    pad_src = (wcol * 997 + jrow * 131) % N
    pad_dst = TRASH + (wcol * 97 + jrow) % NTRASH
    src2 = jnp.concatenate([edge_index[0].reshape(NW, EW), pad_src], axis=1)
    dst2 = jnp.concatenate([edge_index[1].reshape(NW, EW), pad_dst], axis=1)
    s3 = src2.reshape(NW, C, CH)
    d3 = dst2.reshape(NW, C, CH)
    sd = jnp.stack([s3, d3], axis=2).reshape(NW, NGRP, GC, 2, CH)

    xpad = jnp.pad(x, ((0, NPAD - N), (0, 0)))
    batchp = jnp.pad(batch, (0, NPAD - N),
                     constant_values=NG).reshape(1, NPAD)

    ones128 = jnp.ones((CH, D), jnp.float32)
    zeros128 = jnp.zeros((ZROWS, D), jnp.float32)

    degp = _sc_deg(d3, ones128, zeros128)

    h0p = _tc_h0(xpad, W0, degp)
    acc0 = _sc_scatter(h0p, sd, zeros128)
    h1p = _tc_mid(acc0, h0p, degp, b0.reshape(1, D), W1)
    acc1 = _sc_scatter(h1p, sd, zeros128)
    h2p = _tc_mid(acc1, h1p, degp, b1.reshape(1, D), W2)
    acc2 = _sc_scatter(h2p, sd, zeros128)
    hf = _tc_last(acc2, h2p, degp, b2.reshape(1, D))

    y = _tc_poolhead(hf, batchp, fc1_w, fc1_b.reshape(1, NG),
                     fc2_w.reshape(1, NG), fc2_b.reshape(1, 1),
                     out_w.reshape(1, 1), out_b.reshape(1, 1))
    return y[:, :1]


def kernel(x, edge_index, batch, W0, b0, W1, b1, W2, b2,
           fc1_w, fc1_b, fc2_w, fc2_b, out_w, out_b):
    return _run(x, edge_index, batch, W0, b0, W1, b1, W2, b2,
                fc1_w, fc1_b, fc2_w, fc2_b, out_w, out_b)


# speedup vs baseline: 2.7406x; 1.0023x over previous
import functools

import jax
import jax.numpy as jnp
from jax import lax
from jax.experimental import pallas as pl
from jax.experimental.pallas import tpu as pltpu
from jax.experimental.pallas import tpu_sc as plsc

N = 10000
NPAD = 10240
D = 128
E = 320000
NG = 64
NW = 32
EW = E // NW
CH = 128
C = 80
EWPAD = C * CH
TRASH = NPAD
NTRASH = 512
ACC_ROWS = NPAD + NTRASH
ZROWS = ACC_ROWS // 16
WROWS = NPAD // 16

_mesh = plsc.VectorSubcoreMesh(core_axis_name="c", subcore_axis_name="s")


def _deg_body(dst_hbm, ones_hbm, zeros_hbm, deg_out, acc, dsbuf, vones,
              sem0, sem1):
    cc = lax.axis_index("c")
    s = lax.axis_index("s")
    w = s * 2 + cc
    pltpu.sync_copy(zeros_hbm, acc.at[pl.ds(s * ZROWS, ZROWS)])
    pltpu.sync_copy(ones_hbm, vones)
    pltpu.sync_copy(dst_hbm.at[w], dsbuf)
    plsc.subcore_barrier()

    def body2(i, carry):
        c = i * 2
        a1 = pltpu.async_copy(vones, acc.at[dsbuf.at[c]], sem0, add=True)
        a2 = pltpu.async_copy(vones, acc.at[dsbuf.at[c + 1]], sem1, add=True)
        a1.wait()
        a2.wait()
        return carry

    lax.fori_loop(0, C // 2, body2, 0)
    plsc.subcore_barrier()
    pltpu.sync_copy(acc.at[pl.ds(s * WROWS, WROWS)],
                    deg_out.at[cc, pl.ds(s * WROWS, WROWS)])


def _sc_deg(dst3, ones128, zeros128):
    return pl.kernel(
        _deg_body,
        out_type=jax.ShapeDtypeStruct((2, NPAD, D), jnp.float32),
        mesh=_mesh,
        scratch_types=[
            pltpu.VMEM_SHARED((ACC_ROWS, D), jnp.float32),
            pltpu.VMEM((C, CH), jnp.int32),
            pltpu.VMEM((CH, D), jnp.float32),
            pltpu.SemaphoreType.DMA,
            pltpu.SemaphoreType.DMA,
        ],
    )(dst3, ones128, zeros128)


GC = 10
NGRP = C // GC


def _scatter_body(hp_hbm, sd_hbm, zeros_hbm, out, acc, idxb, rows0, rows1,
                  gsem0, gsem1, isem0, isem1, ssem0, ssem1):
    cc = lax.axis_index("c")
    s = lax.axis_index("s")
    w = s * 2 + cc
    isems = (isem0, isem1)
    pltpu.sync_copy(zeros_hbm, acc.at[pl.ds(s * ZROWS, ZROWS)])
    pltpu.sync_copy(sd_hbm.at[w, 0], idxb.at[0])
    plsc.subcore_barrier()

    def gath(par, slot, rbuf, sem):
        return pltpu.async_copy(hp_hbm.at[idxb.at[par, slot, 0]], rbuf, sem)

    def scat(par, slot, rbuf, sem):
        return pltpu.async_copy(rbuf, acc.at[idxb.at[par, slot, 1]], sem,
                                add=True)

    gath(0, 0, rows0, gsem0).wait()

    for g in range(NGRP):
        par = g % 2
        if g + 1 < NGRP:
            ai = pltpu.async_copy(sd_hbm.at[w, g + 1], idxb.at[1 - par],
                                  isems[1 - par])

        def pair(j, carry, par=par):
            e = j * 2
            s_e = scat(par, e, rows0, ssem0)
            g_o = gath(par, e + 1, rows1, gsem1)
            g_o.wait()
            s_o = scat(par, e + 1, rows1, ssem1)
            s_e.wait()
            g_n = gath(par, e + 2, rows0, gsem0)
            s_o.wait()
            g_n.wait()
            return carry

        lax.fori_loop(0, GC // 2 - 1, pair, 0)
        s_e = scat(par, GC - 2, rows0, ssem0)
        g_o = gath(par, GC - 1, rows1, gsem1)
        g_o.wait()
        s_o = scat(par, GC - 1, rows1, ssem1)
        s_e.wait()
        if g + 1 < NGRP:
            ai.wait()
            g_n = gath(1 - par, 0, rows0, gsem0)
        s_o.wait()
        if g + 1 < NGRP:
            g_n.wait()

    plsc.subcore_barrier()
    pltpu.sync_copy(acc.at[pl.ds(s * WROWS, WROWS)],
                    out.at[cc, pl.ds(s * WROWS, WROWS)])


def _sc_scatter(hp, sd, zeros128):
    return pl.kernel(
        _scatter_body,
        out_type=jax.ShapeDtypeStruct((2, NPAD, D), jnp.float32),
        mesh=_mesh,
        scratch_types=[
            pltpu.VMEM_SHARED((ACC_ROWS, D), jnp.float32),
            pltpu.VMEM((2, GC, 2, CH), jnp.int32),
            pltpu.VMEM((CH, D), jnp.float32),
            pltpu.VMEM((CH, D), jnp.float32),
            pltpu.SemaphoreType.DMA,
            pltpu.SemaphoreType.DMA,
            pltpu.SemaphoreType.DMA,
            pltpu.SemaphoreType.DMA,
            pltpu.SemaphoreType.DMA,
            pltpu.SemaphoreType.DMA,
        ],
    )(hp, sd, zeros128)


BLK = 1024
GRID = NPAD // BLK


def _dinv_of(degp):
    deg = degp[0, :, 0] + degp[1, :, 0] + 1.0
    return lax.rsqrt(deg)


def _h0_body(x_ref, w_ref, degp_ref, out_ref):
    dinv = _dinv_of(degp_ref[...])
    h = jnp.dot(x_ref[...], w_ref[...], preferred_element_type=jnp.float32)
    out_ref[...] = h * dinv[:, None]


def _tc_h0(xpad, W0, degp):
    return pl.pallas_call(
        _h0_body,
        grid=(GRID,),
        in_specs=[
            pl.BlockSpec((BLK, D), lambda i: (i, 0)),
            pl.BlockSpec((D, D), lambda i: (0, 0)),
            pl.BlockSpec((2, BLK, D), lambda i: (0, i, 0)),
        ],
        out_specs=pl.BlockSpec((BLK, D), lambda i: (i, 0)),
        out_shape=jax.ShapeDtypeStruct((NPAD, D), jnp.float32),
    )(xpad, W0, degp)


def _mid_body(acc_ref, hp_ref, degp_ref, b_ref, w_ref, out_ref):
    dinv = _dinv_of(degp_ref[...])
    t = dinv[:, None] * (acc_ref[0] + acc_ref[1] + hp_ref[...]) + b_ref[...]
    t = jnp.maximum(t, 0.0)
    h = jnp.dot(t, w_ref[...], preferred_element_type=jnp.float32)
    out_ref[...] = h * dinv[:, None]


def _tc_mid(acc, hp, degp, b, W):
    return pl.pallas_call(
        _mid_body,
        grid=(GRID,),
        in_specs=[
            pl.BlockSpec((2, BLK, D), lambda i: (0, i, 0)),
            pl.BlockSpec((BLK, D), lambda i: (i, 0)),
            pl.BlockSpec((2, BLK, D), lambda i: (0, i, 0)),
            pl.BlockSpec((1, D), lambda i: (0, 0)),
            pl.BlockSpec((D, D), lambda i: (0, 0)),
        ],
        out_specs=pl.BlockSpec((BLK, D), lambda i: (i, 0)),
        out_shape=jax.ShapeDtypeStruct((NPAD, D), jnp.float32),
    )(acc, hp, degp, b, W)


def _last_body(acc_ref, hp_ref, degp_ref, b_ref, out_ref):
    dinv = _dinv_of(degp_ref[...])
    out_ref[...] = (dinv[:, None] * (acc_ref[0] + acc_ref[1] + hp_ref[...])
                    + b_ref[...])


def _tc_last(acc, hp, degp, b):
    return pl.pallas_call(
        _last_body,
        grid=(GRID,),
        in_specs=[
            pl.BlockSpec((2, BLK, D), lambda i: (0, i, 0)),
            pl.BlockSpec((BLK, D), lambda i: (i, 0)),
            pl.BlockSpec((2, BLK, D), lambda i: (0, i, 0)),
            pl.BlockSpec((1, D), lambda i: (0, 0)),
        ],
        out_specs=pl.BlockSpec((BLK, D), lambda i: (i, 0)),
        out_shape=jax.ShapeDtypeStruct((NPAD, D), jnp.float32),
    )(acc, hp, degp, b)


def _poolhead_body(hf_ref, batch_ref, fc1w_ref, fc1b_ref, fc2w_ref,
                   fc2b_ref, outw_ref, outb_ref, y_ref):
    seg = lax.broadcasted_iota(jnp.int32, (NG, NPAD), 0)
    m = (seg == batch_ref[...]).astype(jnp.float32)
    sums = jnp.dot(m, hf_ref[...], preferred_element_type=jnp.float32)
    cnt = jnp.sum(m, axis=1, keepdims=True)
    g = sums / jnp.maximum(cnt, 1.0)
    y1 = jnp.maximum(
        jnp.dot(g, fc1w_ref[...], preferred_element_type=jnp.float32)
        + fc1b_ref[...], 0.0)
    y2 = jnp.sum(y1 * fc2w_ref[...], axis=1, keepdims=True) + fc2b_ref[0, 0]
    y = y2 * outw_ref[0, 0] + outb_ref[0, 0]
    y_ref[...] = jnp.broadcast_to(y, (NG, D))


def _tc_poolhead(hf, batchp, fc1_w, fc1_b, fc2_w, fc2_b, out_w, out_b):
    return pl.pallas_call(
        _poolhead_body,
        out_shape=jax.ShapeDtypeStruct((NG, D), jnp.float32),
    )(hf, batchp, fc1_w, fc1_b, fc2_w, fc2_b, out_w, out_b)


@jax.jit
def _run(x, edge_index, batch, W0, b0, W1, b1, W2, b2,
         fc1_w, fc1_b, fc2_w, fc2_b, out_w, out_b):
    npad = EWPAD - EW
    wcol = jnp.arange(NW, dtype=jnp.int32)[:, None]
    jrow = jnp.arange(npad, dtype=jnp.int32)[None, :]
    pad_src = (wcol * 997 + jrow * 131) % N
    pad_dst = TRASH + (wcol * 97 + jrow) % NTRASH
    src2 = jnp.concatenate([edge_index[0].reshape(NW, EW), pad_src], axis=1)
    dst2 = jnp.concatenate([edge_index[1].reshape(NW, EW), pad_dst], axis=1)
    s3 = src2.reshape(NW, C, CH)
    d3 = dst2.reshape(NW, C, CH)
    sd = jnp.stack([s3, d3], axis=2).reshape(NW, NGRP, GC, 2, CH)

    xpad = jnp.pad(x, ((0, NPAD - N), (0, 0)))
    batchp = jnp.pad(batch, (0, NPAD - N),
                     constant_values=NG).reshape(1, NPAD)

    ones128 = jnp.ones((CH, D), jnp.float32)
    zeros128 = jnp.zeros((ZROWS, D), jnp.float32)

    degp = _sc_deg(d3, ones128, zeros128)

    h0p = _tc_h0(xpad, W0, degp)
    acc0 = _sc_scatter(h0p, sd, zeros128)
    h1p = _tc_mid(acc0, h0p, degp, b0.reshape(1, D), W1)
    acc1 = _sc_scatter(h1p, sd, zeros128)
    h2p = _tc_mid(acc1, h1p, degp, b1.reshape(1, D), W2)
    acc2 = _sc_scatter(h2p, sd, zeros128)
    hf = _tc_last(acc2, h2p, degp, b2.reshape(1, D))

    y = _tc_poolhead(hf, batchp, fc1_w, fc1_b.reshape(1, NG),
                     fc2_w.reshape(1, NG), fc2_b.reshape(1, 1),
                     out_w.reshape(1, 1), out_b.reshape(1, 1))
    return y[:, :1]


def kernel(x, edge_index, batch, W0, b0, W1, b1, W2, b2,
           fc1_w, fc1_b, fc2_w, fc2_b, out_w, out_b):
    return _run(x, edge_index, batch, W0, b0, W1, b1, W2, b2,
                fc1_w, fc1_b, fc2_w, fc2_b, out_w, out_b)


# speedup vs baseline: 2.7592x; 1.0068x over previous
import functools

import jax
import jax.numpy as jnp
from jax import lax
from jax.experimental import pallas as pl
from jax.experimental.pallas import tpu as pltpu
from jax.experimental.pallas import tpu_sc as plsc

N = 10000
NPAD = 10240
D = 128
E = 320000
NG = 64
NW = 32
EW = E // NW
CH = 128
C = 80
EWPAD = C * CH
TRASH = NPAD
NTRASH = 512
ACC_ROWS = NPAD + NTRASH
ZROWS = ACC_ROWS // 16
WROWS = NPAD // 16

_mesh = plsc.VectorSubcoreMesh(core_axis_name="c", subcore_axis_name="s")


def _deg_body(dst_hbm, ones_hbm, zeros_hbm, deg_out, acc, dsbuf, vones,
              sem0, sem1):
    cc = lax.axis_index("c")
    s = lax.axis_index("s")
    w = s * 2 + cc
    pltpu.sync_copy(zeros_hbm, acc.at[pl.ds(s * ZROWS, ZROWS)])
    pltpu.sync_copy(ones_hbm, vones)
    pltpu.sync_copy(dst_hbm.at[w], dsbuf)
    plsc.subcore_barrier()

    def body2(i, carry):
        c = i * 2
        a1 = pltpu.async_copy(vones, acc.at[dsbuf.at[c]], sem0, add=True)
        a2 = pltpu.async_copy(vones, acc.at[dsbuf.at[c + 1]], sem1, add=True)
        a1.wait()
        a2.wait()
        return carry

    lax.fori_loop(0, C // 2, body2, 0)
    plsc.subcore_barrier()
    pltpu.sync_copy(acc.at[pl.ds(s * WROWS, WROWS)],
                    deg_out.at[cc, pl.ds(s * WROWS, WROWS)])


def _sc_deg(dst3, ones128, zeros128):
    return pl.kernel(
        _deg_body,
        out_type=jax.ShapeDtypeStruct((2, NPAD, D), jnp.float32),
        mesh=_mesh,
        scratch_types=[
            pltpu.VMEM_SHARED((ACC_ROWS, D), jnp.float32),
            pltpu.VMEM((C, CH), jnp.int32),
            pltpu.VMEM((CH, D), jnp.float32),
            pltpu.SemaphoreType.DMA,
            pltpu.SemaphoreType.DMA,
        ],
    )(dst3, ones128, zeros128)


GC = 10
NGRP = C // GC


def _scatter_body(hp_hbm, sd_hbm, zeros_hbm, out, acc, idxb, rows0, rows1,
                  gsem0, gsem1, isem0, isem1, ssem0, ssem1):
    cc = lax.axis_index("c")
    s = lax.axis_index("s")
    w = s * 2 + cc
    isems = (isem0, isem1)
    pltpu.sync_copy(zeros_hbm, acc.at[pl.ds(s * ZROWS, ZROWS)])
    pltpu.sync_copy(sd_hbm.at[w, 0], idxb.at[0])
    plsc.subcore_barrier()

    def gath(par, slot, rbuf, sem):
        return pltpu.async_copy(hp_hbm.at[idxb.at[par, slot, 0]], rbuf, sem)

    def scat(par, slot, rbuf, sem):
        return pltpu.async_copy(rbuf, acc.at[idxb.at[par, slot, 1]], sem,
                                add=True)

    gath(0, 0, rows0, gsem0).wait()

    for g in range(NGRP):
        par = g % 2
        if g + 1 < NGRP:
            ai = pltpu.async_copy(sd_hbm.at[w, g + 1], idxb.at[1 - par],
                                  isems[1 - par])

        def pair(j, carry, par=par):
            e = j * 2
            s_e = scat(par, e, rows0, ssem0)
            g_o = gath(par, e + 1, rows1, gsem1)
            g_o.wait()
            s_o = scat(par, e + 1, rows1, ssem1)
            s_e.wait()
            g_n = gath(par, e + 2, rows0, gsem0)
            s_o.wait()
            g_n.wait()
            return carry

        lax.fori_loop(0, GC // 2 - 1, pair, 0)
        s_e = scat(par, GC - 2, rows0, ssem0)
        g_o = gath(par, GC - 1, rows1, gsem1)
        g_o.wait()
        s_o = scat(par, GC - 1, rows1, ssem1)
        s_e.wait()
        if g + 1 < NGRP:
            ai.wait()
            g_n = gath(1 - par, 0, rows0, gsem0)
        s_o.wait()
        if g + 1 < NGRP:
            g_n.wait()

    plsc.subcore_barrier()
    pltpu.sync_copy(acc.at[pl.ds(s * WROWS, WROWS)],
                    out.at[cc, pl.ds(s * WROWS, WROWS)])


def _sc_scatter(hp, sd, zeros128):
    return pl.kernel(
        _scatter_body,
        out_type=jax.ShapeDtypeStruct((2, NPAD, D), jnp.float32),
        mesh=_mesh,
        scratch_types=[
            pltpu.VMEM_SHARED((ACC_ROWS, D), jnp.float32),
            pltpu.VMEM((2, GC, 2, CH), jnp.int32),
            pltpu.VMEM((CH, D), jnp.float32),
            pltpu.VMEM((CH, D), jnp.float32),
            pltpu.SemaphoreType.DMA,
            pltpu.SemaphoreType.DMA,
            pltpu.SemaphoreType.DMA,
            pltpu.SemaphoreType.DMA,
            pltpu.SemaphoreType.DMA,
            pltpu.SemaphoreType.DMA,
        ],
    )(hp, sd, zeros128)


BLK = 1024
GRID = NPAD // BLK


def _dinv_of(degp):
    deg = degp[0, :, 0] + degp[1, :, 0] + 1.0
    return lax.rsqrt(deg)


def _h0_body(x_ref, w_ref, degp_ref, out_ref):
    dinv = _dinv_of(degp_ref[...])
    h = jnp.dot(x_ref[...], w_ref[...], preferred_element_type=jnp.float32)
    out_ref[...] = h * dinv[:, None]


def _tc_h0(xpad, W0, degp):
    return pl.pallas_call(
        _h0_body,
        grid=(GRID,),
        in_specs=[
            pl.BlockSpec((BLK, D), lambda i: (i, 0)),
            pl.BlockSpec((D, D), lambda i: (0, 0)),
            pl.BlockSpec((2, BLK, D), lambda i: (0, i, 0)),
        ],
        out_specs=pl.BlockSpec((BLK, D), lambda i: (i, 0)),
        out_shape=jax.ShapeDtypeStruct((NPAD, D), jnp.float32),
    )(xpad, W0, degp)


def _mid_body(acc_ref, hp_ref, degp_ref, b_ref, w_ref, out_ref):
    dinv = _dinv_of(degp_ref[...])
    t = dinv[:, None] * (acc_ref[0] + acc_ref[1] + hp_ref[...]) + b_ref[...]
    t = jnp.maximum(t, 0.0)
    h = jnp.dot(t, w_ref[...], preferred_element_type=jnp.float32)
    out_ref[...] = h * dinv[:, None]


def _tc_mid(acc, hp, degp, b, W):
    return pl.pallas_call(
        _mid_body,
        grid=(GRID,),
        in_specs=[
            pl.BlockSpec((2, BLK, D), lambda i: (0, i, 0)),
            pl.BlockSpec((BLK, D), lambda i: (i, 0)),
            pl.BlockSpec((2, BLK, D), lambda i: (0, i, 0)),
            pl.BlockSpec((1, D), lambda i: (0, 0)),
            pl.BlockSpec((D, D), lambda i: (0, 0)),
        ],
        out_specs=pl.BlockSpec((BLK, D), lambda i: (i, 0)),
        out_shape=jax.ShapeDtypeStruct((NPAD, D), jnp.float32),
    )(acc, hp, degp, b, W)


def _poolhead_body(acc_ref, hp_ref, degp_ref, b_ref, batch_ref, fc1w_ref,
                   fc1b_ref, fc2w_ref, fc2b_ref, outw_ref, outb_ref, y_ref):
    dinv = _dinv_of(degp_ref[...])
    hf = (dinv[:, None] * (acc_ref[0] + acc_ref[1] + hp_ref[...])
          + b_ref[...])
    seg = lax.broadcasted_iota(jnp.int32, (NG, NPAD), 0)
    m = (seg == batch_ref[...]).astype(jnp.float32)
    sums = jnp.dot(m, hf, preferred_element_type=jnp.float32)
    cnt = jnp.sum(m, axis=1, keepdims=True)
    g = sums / jnp.maximum(cnt, 1.0)
    y1 = jnp.maximum(
        jnp.dot(g, fc1w_ref[...], preferred_element_type=jnp.float32)
        + fc1b_ref[...], 0.0)
    y2 = jnp.sum(y1 * fc2w_ref[...], axis=1, keepdims=True) + fc2b_ref[0, 0]
    y = y2 * outw_ref[0, 0] + outb_ref[0, 0]
    y_ref[...] = jnp.broadcast_to(y, (NG, D))


def _tc_poolhead(acc, hp, degp, b, batchp, fc1_w, fc1_b, fc2_w, fc2_b,
                 out_w, out_b):
    return pl.pallas_call(
        _poolhead_body,
        out_shape=jax.ShapeDtypeStruct((NG, D), jnp.float32),
    )(acc, hp, degp, b, batchp, fc1_w, fc1_b, fc2_w, fc2_b, out_w, out_b)


@jax.jit
def _run(x, edge_index, batch, W0, b0, W1, b1, W2, b2,
         fc1_w, fc1_b, fc2_w, fc2_b, out_w, out_b):
    npad = EWPAD - EW
    wcol = jnp.arange(NW, dtype=jnp.int32)[:, None]
    jrow = jnp.arange(npad, dtype=jnp.int32)[None, :]
    pad_src = (wcol * 997 + jrow * 131) % N
    pad_dst = TRASH + (wcol * 97 + jrow) % NTRASH
    src2 = jnp.concatenate([edge_index[0].reshape(NW, EW), pad_src], axis=1)
    dst2 = jnp.concatenate([edge_index[1].reshape(NW, EW), pad_dst], axis=1)
    s3 = src2.reshape(NW, C, CH)
    d3 = dst2.reshape(NW, C, CH)
    sd = jnp.stack([s3, d3], axis=2).reshape(NW, NGRP, GC, 2, CH)

    xpad = jnp.pad(x, ((0, NPAD - N), (0, 0)))
    batchp = jnp.pad(batch, (0, NPAD - N),
                     constant_values=NG).reshape(1, NPAD)

    ones128 = jnp.ones((CH, D), jnp.float32)
    zeros128 = jnp.zeros((ZROWS, D), jnp.float32)

    degp = _sc_deg(d3, ones128, zeros128)

    h0p = _tc_h0(xpad, W0, degp)
    acc0 = _sc_scatter(h0p, sd, zeros128)
    h1p = _tc_mid(acc0, h0p, degp, b0.reshape(1, D), W1)
    acc1 = _sc_scatter(h1p, sd, zeros128)
    h2p = _tc_mid(acc1, h1p, degp, b1.reshape(1, D), W2)
    acc2 = _sc_scatter(h2p, sd, zeros128)
    y = _tc_poolhead(acc2, h2p, degp, b2.reshape(1, D), batchp,
                     fc1_w, fc1_b.reshape(1, NG),
                     fc2_w.reshape(1, NG), fc2_b.reshape(1, 1),
                     out_w.reshape(1, 1), out_b.reshape(1, 1))
    return y[:, :1]


def kernel(x, edge_index, batch, W0, b0, W1, b1, W2, b2,
           fc1_w, fc1_b, fc2_w, fc2_b, out_w, out_b):
    return _run(x, edge_index, batch, W0, b0, W1, b1, W2, b2,
                fc1_w, fc1_b, fc2_w, fc2_b, out_w, out_b)


# speedup vs baseline: 2.8063x; 1.0171x over previous
import functools

import jax
import jax.numpy as jnp
from jax import lax
from jax.experimental import pallas as pl
from jax.experimental.pallas import tpu as pltpu
from jax.experimental.pallas import tpu_sc as plsc

N = 10000
NPAD = 10240
D = 128
E = 320000
NG = 64
NW = 32
EW = E // NW
CH = 128
C = 80
EWPAD = C * CH
TRASH = NPAD
NTRASH = 512
ACC_ROWS = NPAD + NTRASH
ZROWS = ACC_ROWS // 16
WROWS = NPAD // 16

_mesh = plsc.VectorSubcoreMesh(core_axis_name="c", subcore_axis_name="s")


def _deg_body(dst_hbm, ones_hbm, zeros_hbm, deg_out, acc, dsbuf, vones,
              sem0, sem1):
    cc = lax.axis_index("c")
    s = lax.axis_index("s")
    w = s * 2 + cc
    pltpu.sync_copy(zeros_hbm, acc.at[pl.ds(s * ZROWS, ZROWS)])
    pltpu.sync_copy(ones_hbm, vones)
    pltpu.sync_copy(dst_hbm.at[w], dsbuf)
    plsc.subcore_barrier()

    def body2(i, carry):
        c = i * 2
        a1 = pltpu.async_copy(vones, acc.at[dsbuf.at[c]], sem0, add=True)
        a2 = pltpu.async_copy(vones, acc.at[dsbuf.at[c + 1]], sem1, add=True)
        a1.wait()
        a2.wait()
        return carry

    lax.fori_loop(0, C // 2, body2, 0)
    plsc.subcore_barrier()
    pltpu.sync_copy(acc.at[pl.ds(s * WROWS, WROWS)],
                    deg_out.at[cc, pl.ds(s * WROWS, WROWS)])


def _sc_deg(dst3, ones128, zeros128):
    return pl.kernel(
        _deg_body,
        out_type=jax.ShapeDtypeStruct((2, NPAD, D), jnp.float32),
        mesh=_mesh,
        scratch_types=[
            pltpu.VMEM_SHARED((ACC_ROWS, D), jnp.float32),
            pltpu.VMEM((C, CH), jnp.int32),
            pltpu.VMEM((CH, D), jnp.float32),
            pltpu.SemaphoreType.DMA,
            pltpu.SemaphoreType.DMA,
        ],
    )(dst3, ones128, zeros128)


GC = 10
NGRP = C // GC


def _scatter_body(hp_hbm, sd_hbm, zeros_hbm, out, acc, idxb, rows0, rows1,
                  gsem0, gsem0b, gsem1, gsem1b, isem0, isem1, ssem0, ssem1):
    cc = lax.axis_index("c")
    s = lax.axis_index("s")
    w = s * 2 + cc
    isems = (isem0, isem1)
    pltpu.sync_copy(zeros_hbm, acc.at[pl.ds(s * ZROWS, ZROWS)])
    pltpu.sync_copy(sd_hbm.at[w, 0], idxb.at[0])
    plsc.subcore_barrier()

    def gath(par, slot, rbuf, sems):
        h = CH // 2
        d1 = pltpu.async_copy(
            hp_hbm.at[idxb.at[par, slot, 0, pl.ds(0, h)]],
            rbuf.at[pl.ds(0, h)], sems[0])
        d2 = pltpu.async_copy(
            hp_hbm.at[idxb.at[par, slot, 0, pl.ds(h, h)]],
            rbuf.at[pl.ds(h, h)], sems[1])
        return (d1, d2)

    def gwait(descs):
        descs[0].wait()
        descs[1].wait()

    def scat(par, slot, rbuf, sem):
        return pltpu.async_copy(rbuf, acc.at[idxb.at[par, slot, 1]], sem,
                                add=True)

    gsa = (gsem0, gsem0b)
    gsb = (gsem1, gsem1b)
    gwait(gath(0, 0, rows0, gsa))

    for g in range(NGRP):
        par = g % 2
        if g + 1 < NGRP:
            ai = pltpu.async_copy(sd_hbm.at[w, g + 1], idxb.at[1 - par],
                                  isems[1 - par])

        def pair(j, carry, par=par):
            e = j * 2
            s_e = scat(par, e, rows0, ssem0)
            g_o = gath(par, e + 1, rows1, gsb)
            gwait(g_o)
            s_o = scat(par, e + 1, rows1, ssem1)
            s_e.wait()
            g_n = gath(par, e + 2, rows0, gsa)
            s_o.wait()
            gwait(g_n)
            return carry

        lax.fori_loop(0, GC // 2 - 1, pair, 0)
        s_e = scat(par, GC - 2, rows0, ssem0)
        g_o = gath(par, GC - 1, rows1, gsb)
        gwait(g_o)
        s_o = scat(par, GC - 1, rows1, ssem1)
        s_e.wait()
        if g + 1 < NGRP:
            ai.wait()
            g_n = gath(1 - par, 0, rows0, gsa)
        s_o.wait()
        if g + 1 < NGRP:
            gwait(g_n)

    plsc.subcore_barrier()
    pltpu.sync_copy(acc.at[pl.ds(s * WROWS, WROWS)],
                    out.at[cc, pl.ds(s * WROWS, WROWS)])


def _sc_scatter(hp, sd, zeros128):
    return pl.kernel(
        _scatter_body,
        out_type=jax.ShapeDtypeStruct((2, NPAD, D), jnp.float32),
        mesh=_mesh,
        scratch_types=[
            pltpu.VMEM_SHARED((ACC_ROWS, D), jnp.float32),
            pltpu.VMEM((2, GC, 2, CH), jnp.int32),
            pltpu.VMEM((CH, D), jnp.float32),
            pltpu.VMEM((CH, D), jnp.float32),
            pltpu.SemaphoreType.DMA,
            pltpu.SemaphoreType.DMA,
            pltpu.SemaphoreType.DMA,
            pltpu.SemaphoreType.DMA,
            pltpu.SemaphoreType.DMA,
            pltpu.SemaphoreType.DMA,
            pltpu.SemaphoreType.DMA,
            pltpu.SemaphoreType.DMA,
        ],
    )(hp, sd, zeros128)


BLK = 1024
GRID = NPAD // BLK


def _dinv_of(degp):
    deg = degp[0, :, 0] + degp[1, :, 0] + 1.0
    return lax.rsqrt(deg)


def _h0_body(x_ref, w_ref, degp_ref, out_ref):
    dinv = _dinv_of(degp_ref[...])
    h = jnp.dot(x_ref[...], w_ref[...], preferred_element_type=jnp.float32)
    out_ref[...] = h * dinv[:, None]


def _tc_h0(xpad, W0, degp):
    return pl.pallas_call(
        _h0_body,
        grid=(GRID,),
        in_specs=[
            pl.BlockSpec((BLK, D), lambda i: (i, 0)),
            pl.BlockSpec((D, D), lambda i: (0, 0)),
            pl.BlockSpec((2, BLK, D), lambda i: (0, i, 0)),
        ],
        out_specs=pl.BlockSpec((BLK, D), lambda i: (i, 0)),
        out_shape=jax.ShapeDtypeStruct((NPAD, D), jnp.float32),
    )(xpad, W0, degp)


def _mid_body(acc_ref, hp_ref, degp_ref, b_ref, w_ref, out_ref):
    dinv = _dinv_of(degp_ref[...])
    t = dinv[:, None] * (acc_ref[0] + acc_ref[1] + hp_ref[...]) + b_ref[...]
    t = jnp.maximum(t, 0.0)
    h = jnp.dot(t, w_ref[...], preferred_element_type=jnp.float32)
    out_ref[...] = h * dinv[:, None]


def _tc_mid(acc, hp, degp, b, W):
    return pl.pallas_call(
        _mid_body,
        grid=(GRID,),
        in_specs=[
            pl.BlockSpec((2, BLK, D), lambda i: (0, i, 0)),
            pl.BlockSpec((BLK, D), lambda i: (i, 0)),
            pl.BlockSpec((2, BLK, D), lambda i: (0, i, 0)),
            pl.BlockSpec((1, D), lambda i: (0, 0)),
            pl.BlockSpec((D, D), lambda i: (0, 0)),
        ],
        out_specs=pl.BlockSpec((BLK, D), lambda i: (i, 0)),
        out_shape=jax.ShapeDtypeStruct((NPAD, D), jnp.float32),
    )(acc, hp, degp, b, W)


def _poolhead_body(acc_ref, hp_ref, degp_ref, b_ref, batch_ref, fc1w_ref,
                   fc1b_ref, fc2w_ref, fc2b_ref, outw_ref, outb_ref, y_ref):
    dinv = _dinv_of(degp_ref[...])
    hf = (dinv[:, None] * (acc_ref[0] + acc_ref[1] + hp_ref[...])
          + b_ref[...])
    seg = lax.broadcasted_iota(jnp.int32, (NG, NPAD), 0)
    m = (seg == batch_ref[...]).astype(jnp.float32)
    sums = jnp.dot(m, hf, preferred_element_type=jnp.float32)
    cnt = jnp.sum(m, axis=1, keepdims=True)
    g = sums / jnp.maximum(cnt, 1.0)
    y1 = jnp.maximum(
        jnp.dot(g, fc1w_ref[...], preferred_element_type=jnp.float32)
        + fc1b_ref[...], 0.0)
    y2 = jnp.sum(y1 * fc2w_ref[...], axis=1, keepdims=True) + fc2b_ref[0, 0]
    y = y2 * outw_ref[0, 0] + outb_ref[0, 0]
    y_ref[...] = jnp.broadcast_to(y, (NG, D))


def _tc_poolhead(acc, hp, degp, b, batchp, fc1_w, fc1_b, fc2_w, fc2_b,
                 out_w, out_b):
    return pl.pallas_call(
        _poolhead_body,
        out_shape=jax.ShapeDtypeStruct((NG, D), jnp.float32),
    )(acc, hp, degp, b, batchp, fc1_w, fc1_b, fc2_w, fc2_b, out_w, out_b)


@jax.jit
def _run(x, edge_index, batch, W0, b0, W1, b1, W2, b2,
         fc1_w, fc1_b, fc2_w, fc2_b, out_w, out_b):
    npad = EWPAD - EW
    wcol = jnp.arange(NW, dtype=jnp.int32)[:, None]
    jrow = jnp.arange(npad, dtype=jnp.int32)[None, :]
    pad_src = (wcol * 997 + jrow * 131) % N
    pad_dst = TRASH + (wcol * 97 + jrow) % NTRASH
    src2 = jnp.concatenate([edge_index[0].reshape(NW, EW), pad_src], axis=1)
    dst2 = jnp.concatenate([edge_index[1].reshape(NW, EW), pad_dst], axis=1)
    s3 = src2.reshape(NW, C, CH)
    d3 = dst2.reshape(NW, C, CH)
    sd = jnp.stack([s3, d3], axis=2).reshape(NW, NGRP, GC, 2, CH)

    xpad = jnp.pad(x, ((0, NPAD - N), (0, 0)))
    batchp = jnp.pad(batch, (0, NPAD - N),
                     constant_values=NG).reshape(1, NPAD)

    ones128 = jnp.ones((CH, D), jnp.float32)
    zeros128 = jnp.zeros((ZROWS, D), jnp.float32)

    degp = _sc_deg(d3, ones128, zeros128)

    h0p = _tc_h0(xpad, W0, degp)
    acc0 = _sc_scatter(h0p, sd, zeros128)
    h1p = _tc_mid(acc0, h0p, degp, b0.reshape(1, D), W1)
    acc1 = _sc_scatter(h1p, sd, zeros128)
    h2p = _tc_mid(acc1, h1p, degp, b1.reshape(1, D), W2)
    acc2 = _sc_scatter(h2p, sd, zeros128)
    y = _tc_poolhead(acc2, h2p, degp, b2.reshape(1, D), batchp,
                     fc1_w, fc1_b.reshape(1, NG),
                     fc2_w.reshape(1, NG), fc2_b.reshape(1, 1),
                     out_w.reshape(1, 1), out_b.reshape(1, 1))
    return y[:, :1]


def kernel(x, edge_index, batch, W0, b0, W1, b1, W2, b2,
           fc1_w, fc1_b, fc2_w, fc2_b, out_w, out_b):
    return _run(x, edge_index, batch, W0, b0, W1, b1, W2, b2,
                fc1_w, fc1_b, fc2_w, fc2_b, out_w, out_b)


# speedup vs baseline: 2.8142x; 1.0028x over previous
import functools

import jax
import jax.numpy as jnp
from jax import lax
from jax.experimental import pallas as pl
from jax.experimental.pallas import tpu as pltpu
from jax.experimental.pallas import tpu_sc as plsc

N = 10000
NPAD = 10240
D = 128
E = 320000
NG = 64
NW = 32
EW = E // NW
CH = 128
C = 80
EWPAD = C * CH
TRASH = NPAD
NTRASH = 512
ACC_ROWS = NPAD + NTRASH
ZROWS = ACC_ROWS // 16
WROWS = NPAD // 16

_mesh = plsc.VectorSubcoreMesh(core_axis_name="c", subcore_axis_name="s")


def _deg_body(dst_hbm, ones_hbm, zeros_hbm, deg_out, acc, dsbuf, vones,
              sem0, sem1):
    cc = lax.axis_index("c")
    s = lax.axis_index("s")
    w = s * 2 + cc
    pltpu.sync_copy(zeros_hbm, acc.at[pl.ds(s * ZROWS, ZROWS)])
    pltpu.sync_copy(ones_hbm, vones)
    pltpu.sync_copy(dst_hbm.at[w], dsbuf)
    plsc.subcore_barrier()

    def body2(i, carry):
        c = i * 2
        a1 = pltpu.async_copy(vones, acc.at[dsbuf.at[c]], sem0, add=True)
        a2 = pltpu.async_copy(vones, acc.at[dsbuf.at[c + 1]], sem1, add=True)
        a1.wait()
        a2.wait()
        return carry

    lax.fori_loop(0, C // 2, body2, 0)
    plsc.subcore_barrier()
    pltpu.sync_copy(acc.at[pl.ds(s * WROWS, WROWS)],
                    deg_out.at[cc, pl.ds(s * WROWS, WROWS)])


def _sc_deg(dst3, ones128, zeros128):
    return pl.kernel(
        _deg_body,
        out_type=jax.ShapeDtypeStruct((2, NPAD, D), jnp.float32),
        mesh=_mesh,
        scratch_types=[
            pltpu.VMEM_SHARED((ACC_ROWS, D), jnp.float32),
            pltpu.VMEM((C, CH), jnp.int32),
            pltpu.VMEM((CH, D), jnp.float32),
            pltpu.SemaphoreType.DMA,
            pltpu.SemaphoreType.DMA,
        ],
    )(dst3, ones128, zeros128)


GC = 10
NGRP = C // GC


def _scatter_body(hp_hbm, sd_hbm, zeros_hbm, out, acc, idxb, rows0, rows1,
                  gsem0, gsem0b, gsem1, gsem1b, isem0, isem1, ssem0, ssem1):
    cc = lax.axis_index("c")
    s = lax.axis_index("s")
    w = s * 2 + cc
    isems = (isem0, isem1)
    pltpu.sync_copy(zeros_hbm, acc.at[pl.ds(s * ZROWS, ZROWS)])
    pltpu.sync_copy(sd_hbm.at[w, 0], idxb.at[0])
    plsc.subcore_barrier()

    def gath(par, slot, rbuf, sems):
        h = CH // 2
        d1 = pltpu.async_copy(
            hp_hbm.at[idxb.at[par, slot, 0, pl.ds(0, h)]],
            rbuf.at[pl.ds(0, h)], sems[0])
        d2 = pltpu.async_copy(
            hp_hbm.at[idxb.at[par, slot, 0, pl.ds(h, h)]],
            rbuf.at[pl.ds(h, h)], sems[1])
        return (d1, d2)

    def gwait(descs):
        descs[0].wait()
        descs[1].wait()

    def scat(par, slot, rbuf, sem):
        return pltpu.async_copy(rbuf, acc.at[idxb.at[par, slot, 1]], sem,
                                add=True)

    gsa = (gsem0, gsem0b)
    gsb = (gsem1, gsem1b)
    gwait(gath(0, 0, rows0, gsa))

    for g in range(NGRP):
        par = g % 2
        if g + 1 < NGRP:
            ai = pltpu.async_copy(sd_hbm.at[w, g + 1], idxb.at[1 - par],
                                  isems[1 - par])

        def pair(j, carry, par=par):
            e = j * 2
            s_e = scat(par, e, rows0, ssem0)
            g_o = gath(par, e + 1, rows1, gsb)
            gwait(g_o)
            s_o = scat(par, e + 1, rows1, ssem1)
            s_e.wait()
            g_n = gath(par, e + 2, rows0, gsa)
            s_o.wait()
            gwait(g_n)
            return carry

        lax.fori_loop(0, GC // 2 - 1, pair, 0)
        s_e = scat(par, GC - 2, rows0, ssem0)
        g_o = gath(par, GC - 1, rows1, gsb)
        gwait(g_o)
        s_o = scat(par, GC - 1, rows1, ssem1)
        s_e.wait()
        if g + 1 < NGRP:
            ai.wait()
            g_n = gath(1 - par, 0, rows0, gsa)
        s_o.wait()
        if g + 1 < NGRP:
            gwait(g_n)

    plsc.subcore_barrier()
    pltpu.sync_copy(acc.at[pl.ds(s * WROWS, WROWS)],
                    out.at[cc, pl.ds(s * WROWS, WROWS)])


def _sc_scatter(hp, sd, zeros128):
    return pl.kernel(
        _scatter_body,
        out_type=jax.ShapeDtypeStruct((2, NPAD, D), jnp.float32),
        mesh=_mesh,
        scratch_types=[
            pltpu.VMEM_SHARED((ACC_ROWS, D), jnp.float32),
            pltpu.VMEM((2, GC, 2, CH), jnp.int32),
            pltpu.VMEM((CH, D), jnp.float32),
            pltpu.VMEM((CH, D), jnp.float32),
            pltpu.SemaphoreType.DMA,
            pltpu.SemaphoreType.DMA,
            pltpu.SemaphoreType.DMA,
            pltpu.SemaphoreType.DMA,
            pltpu.SemaphoreType.DMA,
            pltpu.SemaphoreType.DMA,
            pltpu.SemaphoreType.DMA,
            pltpu.SemaphoreType.DMA,
        ],
    )(hp, sd, zeros128)


BLK = 1024
GRID = NPAD // BLK


def _dinv_of(degp):
    deg = degp[0, :, 0] + degp[1, :, 0] + 1.0
    return lax.rsqrt(deg)


def _h0_body(x_ref, w_ref, degp_ref, out_ref, dinv_ref):
    dinv = _dinv_of(degp_ref[...])
    h = jnp.dot(x_ref[...], w_ref[...], preferred_element_type=jnp.float32)
    out_ref[...] = h * dinv[:, None]
    dinv_ref[...] = jnp.broadcast_to(dinv[:, None], (BLK, 8))


def _tc_h0(xpad, W0, degp):
    return pl.pallas_call(
        _h0_body,
        grid=(GRID,),
        in_specs=[
            pl.BlockSpec((BLK, D), lambda i: (i, 0)),
            pl.BlockSpec((D, D), lambda i: (0, 0)),
            pl.BlockSpec((2, BLK, D), lambda i: (0, i, 0)),
        ],
        out_specs=[
            pl.BlockSpec((BLK, D), lambda i: (i, 0)),
            pl.BlockSpec((BLK, 8), lambda i: (i, 0)),
        ],
        out_shape=[
            jax.ShapeDtypeStruct((NPAD, D), jnp.float32),
            jax.ShapeDtypeStruct((NPAD, 8), jnp.float32),
        ],
    )(xpad, W0, degp)


def _mid_body(acc_ref, hp_ref, dinv_ref, b_ref, w_ref, out_ref):
    dinv = dinv_ref[:, 0]
    t = dinv[:, None] * (acc_ref[0] + acc_ref[1] + hp_ref[...]) + b_ref[...]
    t = jnp.maximum(t, 0.0)
    h = jnp.dot(t, w_ref[...], preferred_element_type=jnp.float32)
    out_ref[...] = h * dinv[:, None]


def _tc_mid(acc, hp, dinv8, b, W):
    return pl.pallas_call(
        _mid_body,
        grid=(GRID,),
        in_specs=[
            pl.BlockSpec((2, BLK, D), lambda i: (0, i, 0)),
            pl.BlockSpec((BLK, D), lambda i: (i, 0)),
            pl.BlockSpec((BLK, 8), lambda i: (i, 0)),
            pl.BlockSpec((1, D), lambda i: (0, 0)),
            pl.BlockSpec((D, D), lambda i: (0, 0)),
        ],
        out_specs=pl.BlockSpec((BLK, D), lambda i: (i, 0)),
        out_shape=jax.ShapeDtypeStruct((NPAD, D), jnp.float32),
    )(acc, hp, dinv8, b, W)


def _poolhead_body(acc_ref, hp_ref, dinv_ref, b_ref, batch_ref, fc1w_ref,
                   fc1b_ref, fc2w_ref, fc2b_ref, outw_ref, outb_ref, y_ref):
    dinv = dinv_ref[:, 0]
    hf = (dinv[:, None] * (acc_ref[0] + acc_ref[1] + hp_ref[...])
          + b_ref[...])
    seg = lax.broadcasted_iota(jnp.int32, (NG, NPAD), 0)
    m = (seg == batch_ref[...]).astype(jnp.float32)
    sums = jnp.dot(m, hf, preferred_element_type=jnp.float32)
    cnt = jnp.sum(m, axis=1, keepdims=True)
    g = sums / jnp.maximum(cnt, 1.0)
    y1 = jnp.maximum(
        jnp.dot(g, fc1w_ref[...], preferred_element_type=jnp.float32)
        + fc1b_ref[...], 0.0)
    y2 = jnp.sum(y1 * fc2w_ref[...], axis=1, keepdims=True) + fc2b_ref[0, 0]
    y = y2 * outw_ref[0, 0] + outb_ref[0, 0]
    y_ref[...] = jnp.broadcast_to(y, (NG, D))


def _tc_poolhead(acc, hp, dinv8, b, batchp, fc1_w, fc1_b, fc2_w, fc2_b,
                 out_w, out_b):
    return pl.pallas_call(
        _poolhead_body,
        out_shape=jax.ShapeDtypeStruct((NG, D), jnp.float32),
    )(acc, hp, dinv8, b, batchp, fc1_w, fc1_b, fc2_w, fc2_b, out_w, out_b)


@jax.jit
def _run(x, edge_index, batch, W0, b0, W1, b1, W2, b2,
         fc1_w, fc1_b, fc2_w, fc2_b, out_w, out_b):
    npad = EWPAD - EW
    wcol = jnp.arange(NW, dtype=jnp.int32)[:, None]
    jrow = jnp.arange(npad, dtype=jnp.int32)[None, :]
    pad_src = (wcol * 997 + jrow * 131) % N
    pad_dst = TRASH + (wcol * 97 + jrow) % NTRASH
    src2 = jnp.concatenate([edge_index[0].reshape(NW, EW), pad_src], axis=1)
    dst2 = jnp.concatenate([edge_index[1].reshape(NW, EW), pad_dst], axis=1)
    s3 = src2.reshape(NW, C, CH)
    d3 = dst2.reshape(NW, C, CH)
    sd = jnp.stack([s3, d3], axis=2).reshape(NW, NGRP, GC, 2, CH)

    xpad = jnp.pad(x, ((0, NPAD - N), (0, 0)))
    batchp = jnp.pad(batch, (0, NPAD - N),
                     constant_values=NG).reshape(1, NPAD)

    ones128 = jnp.ones((CH, D), jnp.float32)
    zeros128 = jnp.zeros((ZROWS, D), jnp.float32)

    degp = _sc_deg(d3, ones128, zeros128)

    h0p, dinv8 = _tc_h0(xpad, W0, degp)
    acc0 = _sc_scatter(h0p, sd, zeros128)
    h1p = _tc_mid(acc0, h0p, dinv8, b0.reshape(1, D), W1)
    acc1 = _sc_scatter(h1p, sd, zeros128)
    h2p = _tc_mid(acc1, h1p, dinv8, b1.reshape(1, D), W2)
    acc2 = _sc_scatter(h2p, sd, zeros128)
    y = _tc_poolhead(acc2, h2p, dinv8, b2.reshape(1, D), batchp,
                     fc1_w, fc1_b.reshape(1, NG),
                     fc2_w.reshape(1, NG), fc2_b.reshape(1, 1),
                     out_w.reshape(1, 1), out_b.reshape(1, 1))
    return y[:, :1]


def kernel(x, edge_index, batch, W0, b0, W1, b1, W2, b2,
           fc1_w, fc1_b, fc2_w, fc2_b, out_w, out_b):
    return _run(x, edge_index, batch, W0, b0, W1, b1, W2, b2,
                fc1_w, fc1_b, fc2_w, fc2_b, out_w, out_b)
